# baseline (speedup 1.0000x reference)
import functools

import jax
import jax.numpy as jnp
from jax import lax
from jax.experimental import pallas as pl
from jax.experimental.pallas import tpu as pltpu
from jax.experimental.pallas import tpu_sc as plsc

E = 16
B = 64
C = 384
HW = 196
BB = 8
NSUB = 16
RPS = B // NSUB
NEG = -3.0e38


def _logits_body(cw_ref, cb_ref, x_ref, m_ref, gw_ref, gb_ref, nw_ref,
                 nb_ref, eps_ref, out_ref):
    scale = cw_ref[0, 0]
    bias = cb_ref[0, 0]
    w = m_ref[...] * scale + (1.0 + bias)
    x2 = x_ref[...].reshape(BB * C, HW)
    y = lax.dot_general(x2, w, (((1,), (1,)), ((), ())),
                        preferred_element_type=jnp.float32,
                        precision=lax.Precision.HIGHEST)
    y3 = y.reshape(BB, C, BB)
    bi = lax.broadcasted_iota(jnp.int32, (BB, 1, BB), 0)
    bj = lax.broadcasted_iota(jnp.int32, (BB, 1, BB), 2)
    pooled = jnp.sum(jnp.where(bi == bj, y3, 0.0), axis=2) * (1.0 / HW)
    clean = jnp.dot(pooled, gw_ref[...],
                    preferred_element_type=jnp.float32,
                    precision=lax.Precision.HIGHEST) + gb_ref[...]
    raw = jnp.dot(pooled, nw_ref[...],
                  preferred_element_type=jnp.float32,
                  precision=lax.Precision.HIGHEST) + nb_ref[...]
    std = jnp.log(1.0 + jnp.exp(-jnp.abs(raw))) + jnp.maximum(raw, 0.0) + 0.01
    out_ref[...] = clean + eps_ref[...] * std


def _bcast_argmax(v, iota):
    val, idx = v, iota
    for sh in (8, 4, 2, 1):
        perm = jnp.bitwise_xor(iota, sh)
        oval = jnp.take_along_axis(val, perm, axis=0)
        oidx = jnp.take_along_axis(idx, perm, axis=0)
        take = (oval > val) | ((oval == val) & (oidx < idx))
        val = jnp.where(take, oval, val)
        idx = jnp.where(take, oidx, idx)
    return val, idx


def _route_row(v, iota):
    m1, i1 = _bcast_argmax(v, iota)
    is1 = iota == i1
    v2 = jnp.where(is1, NEG, v)
    m2, i2 = _bcast_argmax(v2, iota)
    is2 = iota == i2
    e = jnp.exp(m2 - m1)
    g1 = 1.0 / (1.0 + e)
    return jnp.where(is1, g1, jnp.where(is2, 1.0 - g1, 0.0))


def _route_body(logits_hbm, gates_hbm, load_hbm, rows_v, out_v, all_v,
                loadout_v):
    s = lax.axis_index("s")
    base = s * RPS
    pltpu.sync_copy(logits_hbm.at[pl.ds(base, RPS)], rows_v)
    iota = lax.iota(jnp.int32, 16)
    for r in range(RPS):
        out_v[r] = _route_row(rows_v[r], iota)
    pltpu.sync_copy(out_v, gates_hbm.at[pl.ds(base, RPS)])

    @pl.when(s == 0)
    def _():
        pltpu.sync_copy(logits_hbm, all_v)
        acc = jnp.zeros((16,), jnp.float32)
        for b in range(B):
            acc = acc + _route_row(all_v[b], iota)
        loadout_v[...] = acc
        pltpu.sync_copy(loadout_v, load_hbm)


@functools.cache
def _route():
    return pl.kernel(
        _route_body,
        out_type=(jax.ShapeDtypeStruct((B, E), jnp.float32),
                  jax.ShapeDtypeStruct((E,), jnp.float32)),
        mesh=plsc.VectorSubcoreMesh(core_axis_name="c", subcore_axis_name="s",
                                    num_cores=1, num_subcores=NSUB),
        scratch_types=[
            pltpu.VMEM((RPS, 16), jnp.float32),
            pltpu.VMEM((RPS, 16), jnp.float32),
            pltpu.VMEM((B, 16), jnp.float32),
            pltpu.VMEM((16,), jnp.float32),
        ],
    )


def kernel(x, m, conv_w, conv_b, gate_w, gate_b, noise_w, noise_b):
    x3 = x.reshape(B, C, HW)
    m2 = m.reshape(B, HW)
    cw = conv_w.reshape(1, 1)
    cb = conv_b.reshape(1, 1)
    gwT = gate_w.T
    nwT = noise_w.T
    gb = gate_b.reshape(1, E)
    nb = noise_b.reshape(1, E)
    eps = jax.random.normal(jax.random.key(1234), (B, E), jnp.float32)
    logits = pl.pallas_call(
        _logits_body,
        grid=(B // BB,),
        in_specs=[
            pl.BlockSpec(memory_space=pltpu.SMEM),
            pl.BlockSpec(memory_space=pltpu.SMEM),
            pl.BlockSpec((BB, C, HW), lambda i: (i, 0, 0)),
            pl.BlockSpec((BB, HW), lambda i: (i, 0)),
            pl.BlockSpec((C, E), lambda i: (0, 0)),
            pl.BlockSpec((1, E), lambda i: (0, 0)),
            pl.BlockSpec((C, E), lambda i: (0, 0)),
            pl.BlockSpec((1, E), lambda i: (0, 0)),
            pl.BlockSpec((BB, E), lambda i: (i, 0)),
        ],
        out_specs=pl.BlockSpec((BB, E), lambda i: (i, 0)),
        out_shape=jax.ShapeDtypeStruct((B, E), jnp.float32),
    )(cw, cb, x3, m2, gwT, gb, nwT, nb, eps)
    gates, load = _route()(logits)
    return gates, load


# speedup vs baseline: 2.3927x; 2.3927x over previous
import functools

import jax
import jax.numpy as jnp
from jax import lax
from jax.experimental import pallas as pl
from jax.experimental.pallas import tpu as pltpu
from jax.experimental.pallas import tpu_sc as plsc

E = 16
B = 64
C = 384
HW = 196
BB = 8
NSUB = 16
RPS = B // NSUB
NEG = -3.0e38


CHUNK = 28
NSTEPS = HW // CHUNK


def _logits_body(cw_ref, cb_ref, xt_ref, mt_ref, gw_ref, gb_ref, nw_ref,
                 nb_ref, eps_ref, out_ref, acc_ref):
    step = pl.program_id(0)
    scale = cw_ref[0, 0]
    bias = cb_ref[0, 0]
    wt = jnp.transpose(mt_ref[...], (0, 2, 1)) * scale + (1.0 + bias)
    part = jnp.sum(xt_ref[...] * wt, axis=0)

    @pl.when(step == 0)
    def _():
        acc_ref[...] = part

    @pl.when(step > 0)
    def _():
        acc_ref[...] += part

    @pl.when(step == NSTEPS - 1)
    def _():
        pooled = acc_ref[...] * (1.0 / HW)
        nt = (((1,), (1,)), ((), ()))
        clean = lax.dot_general(pooled, gw_ref[...], nt,
                                preferred_element_type=jnp.float32,
                                precision=lax.Precision.HIGHEST) + gb_ref[...]
        raw = lax.dot_general(pooled, nw_ref[...], nt,
                              preferred_element_type=jnp.float32,
                              precision=lax.Precision.HIGHEST) + nb_ref[...]
        std = (jnp.log(1.0 + jnp.exp(-jnp.abs(raw)))
               + jnp.maximum(raw, 0.0) + 0.01)
        out_ref[...] = clean + eps_ref[...] * std


def _bcast_argmax(v, iota):
    val, idx = v, iota
    for sh in (8, 4, 2, 1):
        perm = jnp.bitwise_xor(iota, sh)
        oval = jnp.take_along_axis(val, perm, axis=0)
        oidx = jnp.take_along_axis(idx, perm, axis=0)
        take = (oval > val) | ((oval == val) & (oidx < idx))
        val = jnp.where(take, oval, val)
        idx = jnp.where(take, oidx, idx)
    return val, idx


def _route_row(v, iota):
    m1, i1 = _bcast_argmax(v, iota)
    is1 = iota == i1
    v2 = jnp.where(is1, NEG, v)
    m2, i2 = _bcast_argmax(v2, iota)
    is2 = iota == i2
    e = jnp.exp(m2 - m1)
    g1 = 1.0 / (1.0 + e)
    return jnp.where(is1, g1, jnp.where(is2, 1.0 - g1, 0.0))


def _route_body(logits_hbm, gates_hbm, load_hbm, rows_v, out_v, all_v,
                loadout_v):
    s = lax.axis_index("s")
    base = s * RPS
    pltpu.sync_copy(logits_hbm.at[pl.ds(base, RPS)], rows_v)
    iota = lax.iota(jnp.int32, 16)
    for r in range(RPS):
        out_v[r] = _route_row(rows_v[r], iota)
    pltpu.sync_copy(out_v, gates_hbm.at[pl.ds(base, RPS)])

    @pl.when(s == 0)
    def _():
        pltpu.sync_copy(logits_hbm, all_v)
        acc = jnp.zeros((16,), jnp.float32)
        for b in range(B):
            acc = acc + _route_row(all_v[b], iota)
        loadout_v[...] = acc
        pltpu.sync_copy(loadout_v, load_hbm)


@functools.cache
def _route():
    return pl.kernel(
        _route_body,
        out_type=(jax.ShapeDtypeStruct((B, E), jnp.float32),
                  jax.ShapeDtypeStruct((E,), jnp.float32)),
        mesh=plsc.VectorSubcoreMesh(core_axis_name="c", subcore_axis_name="s",
                                    num_cores=1, num_subcores=NSUB),
        scratch_types=[
            pltpu.VMEM((RPS, 16), jnp.float32),
            pltpu.VMEM((RPS, 16), jnp.float32),
            pltpu.VMEM((B, 16), jnp.float32),
            pltpu.VMEM((16,), jnp.float32),
        ],
    )


def kernel(x, m, conv_w, conv_b, gate_w, gate_b, noise_w, noise_b):
    xt = jnp.transpose(x, (2, 3, 0, 1)).reshape(HW, B, C)
    mt = jnp.transpose(m, (2, 3, 0, 1)).reshape(HW, 1, B)
    cw = conv_w.reshape(1, 1)
    cb = conv_b.reshape(1, 1)
    gb = gate_b.reshape(1, E)
    nb = noise_b.reshape(1, E)
    eps = jax.random.normal(jax.random.key(1234), (B, E), jnp.float32)
    logits = pl.pallas_call(
        _logits_body,
        grid=(NSTEPS,),
        in_specs=[
            pl.BlockSpec(memory_space=pltpu.SMEM),
            pl.BlockSpec(memory_space=pltpu.SMEM),
            pl.BlockSpec((CHUNK, B, C), lambda i: (i, 0, 0)),
            pl.BlockSpec((CHUNK, 1, B), lambda i: (i, 0, 0)),
            pl.BlockSpec((E, C), lambda i: (0, 0)),
            pl.BlockSpec((1, E), lambda i: (0, 0)),
            pl.BlockSpec((E, C), lambda i: (0, 0)),
            pl.BlockSpec((1, E), lambda i: (0, 0)),
            pl.BlockSpec((B, E), lambda i: (0, 0)),
        ],
        out_specs=pl.BlockSpec((B, E), lambda i: (0, 0)),
        out_shape=jax.ShapeDtypeStruct((B, E), jnp.float32),
        scratch_shapes=[pltpu.VMEM((B, C), jnp.float32)],
    )(cw, cb, xt, mt, gate_w, gb, noise_w, nb, eps)
    gates, load = _route()(logits)
    return gates, load


# speedup vs baseline: 5.3234x; 2.2249x over previous
import functools

import jax
import jax.numpy as jnp
from jax import lax
from jax.experimental import pallas as pl
from jax.experimental.pallas import tpu as pltpu
from jax.experimental.pallas import tpu_sc as plsc

E = 16
B = 64
C = 384
HW = 196
BB = 8
NSUB = 16
RPS = B // NSUB
NEG = -3.0e38


CHUNK = 28
NSTEPS = HW // CHUNK


def _logits_body(cw_ref, cb_ref, xt_ref, mt_ref, gw_ref, gb_ref, nw_ref,
                 nb_ref, eps_ref, out_ref, load_ref, acc_ref):
    step = pl.program_id(0)
    scale = cw_ref[0, 0]
    bias = cb_ref[0, 0]
    wt = jnp.transpose(mt_ref[...], (0, 2, 1)) * scale + (1.0 + bias)
    part = jnp.sum(xt_ref[...] * wt, axis=0)

    @pl.when(step == 0)
    def _():
        acc_ref[...] = part

    @pl.when(step > 0)
    def _():
        acc_ref[...] += part

    @pl.when(step == NSTEPS - 1)
    def _():
        pooled = acc_ref[...] * (1.0 / HW)
        nt = (((1,), (1,)), ((), ()))
        clean = lax.dot_general(pooled, gw_ref[...], nt,
                                preferred_element_type=jnp.float32,
                                precision=lax.Precision.HIGHEST) + gb_ref[...]
        raw = lax.dot_general(pooled, nw_ref[...], nt,
                              preferred_element_type=jnp.float32,
                              precision=lax.Precision.HIGHEST) + nb_ref[...]
        std = (jnp.log(1.0 + jnp.exp(-jnp.abs(raw)))
               + jnp.maximum(raw, 0.0) + 0.01)
        logits = clean + eps_ref[...] * std
        ii = lax.broadcasted_iota(jnp.int32, (B, E), 1)
        m1 = jnp.max(logits, axis=1, keepdims=True)
        i1 = jnp.min(jnp.where(logits == m1, ii, E), axis=1, keepdims=True)
        is1 = ii == i1
        v2 = jnp.where(is1, NEG, logits)
        m2 = jnp.max(v2, axis=1, keepdims=True)
        i2 = jnp.min(jnp.where(v2 == m2, ii, E), axis=1, keepdims=True)
        is2 = ii == i2
        e = jnp.exp(m2 - m1)
        g1 = 1.0 / (1.0 + e)
        gates = jnp.where(is1, g1, jnp.where(is2, 1.0 - g1, 0.0))
        out_ref[...] = gates
        load_ref[...] = jnp.sum(gates, axis=0, keepdims=True)


def _bcast_argmax(v, iota):
    val, idx = v, iota
    for sh in (8, 4, 2, 1):
        perm = jnp.bitwise_xor(iota, sh)
        oval = jnp.take_along_axis(val, perm, axis=0)
        oidx = jnp.take_along_axis(idx, perm, axis=0)
        take = (oval > val) | ((oval == val) & (oidx < idx))
        val = jnp.where(take, oval, val)
        idx = jnp.where(take, oidx, idx)
    return val, idx


def _route_row(v, iota):
    m1, i1 = _bcast_argmax(v, iota)
    is1 = iota == i1
    v2 = jnp.where(is1, NEG, v)
    m2, i2 = _bcast_argmax(v2, iota)
    is2 = iota == i2
    e = jnp.exp(m2 - m1)
    g1 = 1.0 / (1.0 + e)
    return jnp.where(is1, g1, jnp.where(is2, 1.0 - g1, 0.0))


def _route_body(logits_hbm, gates_hbm, load_hbm, rows_v, out_v, all_v,
                loadout_v):
    s = lax.axis_index("s")
    base = s * RPS
    pltpu.sync_copy(logits_hbm.at[pl.ds(base, RPS)], rows_v)
    iota = lax.iota(jnp.int32, 16)
    for r in range(RPS):
        out_v[r] = _route_row(rows_v[r], iota)
    pltpu.sync_copy(out_v, gates_hbm.at[pl.ds(base, RPS)])

    @pl.when(s == 0)
    def _():
        pltpu.sync_copy(logits_hbm, all_v)
        acc = jnp.zeros((16,), jnp.float32)
        for b in range(B):
            acc = acc + _route_row(all_v[b], iota)
        loadout_v[...] = acc
        pltpu.sync_copy(loadout_v, load_hbm)


@functools.cache
def _route():
    return pl.kernel(
        _route_body,
        out_type=(jax.ShapeDtypeStruct((B, E), jnp.float32),
                  jax.ShapeDtypeStruct((E,), jnp.float32)),
        mesh=plsc.VectorSubcoreMesh(core_axis_name="c", subcore_axis_name="s",
                                    num_cores=1, num_subcores=NSUB),
        scratch_types=[
            pltpu.VMEM((RPS, 16), jnp.float32),
            pltpu.VMEM((RPS, 16), jnp.float32),
            pltpu.VMEM((B, 16), jnp.float32),
            pltpu.VMEM((16,), jnp.float32),
        ],
    )


def kernel(x, m, conv_w, conv_b, gate_w, gate_b, noise_w, noise_b):
    xt = jnp.transpose(x, (2, 3, 0, 1)).reshape(HW, B, C)
    mt = jnp.transpose(m, (2, 3, 0, 1)).reshape(HW, 1, B)
    cw = conv_w.reshape(1, 1)
    cb = conv_b.reshape(1, 1)
    gb = gate_b.reshape(1, E)
    nb = noise_b.reshape(1, E)
    eps = jax.random.normal(jax.random.key(1234), (B, E), jnp.float32)
    logits = pl.pallas_call(
        _logits_body,
        grid=(NSTEPS,),
        in_specs=[
            pl.BlockSpec(memory_space=pltpu.SMEM),
            pl.BlockSpec(memory_space=pltpu.SMEM),
            pl.BlockSpec((CHUNK, B, C), lambda i: (i, 0, 0)),
            pl.BlockSpec((CHUNK, 1, B), lambda i: (i, 0, 0)),
            pl.BlockSpec((E, C), lambda i: (0, 0)),
            pl.BlockSpec((1, E), lambda i: (0, 0)),
            pl.BlockSpec((E, C), lambda i: (0, 0)),
            pl.BlockSpec((1, E), lambda i: (0, 0)),
            pl.BlockSpec((B, E), lambda i: (0, 0)),
        ],
        out_specs=[pl.BlockSpec((B, E), lambda i: (0, 0)),
                   pl.BlockSpec((1, E), lambda i: (0, 0))],
        out_shape=[jax.ShapeDtypeStruct((B, E), jnp.float32),
                   jax.ShapeDtypeStruct((1, E), jnp.float32)],
        scratch_shapes=[pltpu.VMEM((B, C), jnp.float32)],
    )(cw, cb, xt, mt, gate_w, gb, noise_w, nb, eps)
    gates, load2 = logits
    return gates, load2.reshape(E)


# speedup vs baseline: 5.9391x; 1.1157x over previous
import functools

import jax
import jax.numpy as jnp
from jax import lax
from jax.experimental import pallas as pl
from jax.experimental.pallas import tpu as pltpu
from jax.experimental.pallas import tpu_sc as plsc

E = 16
B = 64
C = 384
HW = 196
BB = 8
NSUB = 16
RPS = B // NSUB
NEG = -3.0e38


CHUNK = 49
NSTEPS = HW // CHUNK


def _logits_body(cw_ref, cb_ref, xt_ref, mt_ref, gw_ref, gb_ref, nw_ref,
                 nb_ref, eps_ref, out_ref, load_ref, acc_ref):
    step = pl.program_id(0)
    scale = cw_ref[0, 0]
    bias = cb_ref[0, 0]
    wt = jnp.transpose(mt_ref[...], (0, 2, 1)) * scale + (1.0 + bias)
    part = jnp.sum(xt_ref[...] * wt, axis=0)

    @pl.when(step == 0)
    def _():
        acc_ref[...] = part

    @pl.when(step > 0)
    def _():
        acc_ref[...] += part

    @pl.when(step == NSTEPS - 1)
    def _():
        pooled = acc_ref[...] * (1.0 / HW)
        nt = (((1,), (1,)), ((), ()))
        clean = lax.dot_general(pooled, gw_ref[...], nt,
                                preferred_element_type=jnp.float32,
                                precision=lax.Precision.HIGHEST) + gb_ref[...]
        raw = lax.dot_general(pooled, nw_ref[...], nt,
                              preferred_element_type=jnp.float32,
                              precision=lax.Precision.HIGHEST) + nb_ref[...]
        std = (jnp.log(1.0 + jnp.exp(-jnp.abs(raw)))
               + jnp.maximum(raw, 0.0) + 0.01)
        logits = clean + eps_ref[...] * std
        ii = lax.broadcasted_iota(jnp.int32, (B, E), 1)
        m1 = jnp.max(logits, axis=1, keepdims=True)
        i1 = jnp.min(jnp.where(logits == m1, ii, E), axis=1, keepdims=True)
        is1 = ii == i1
        v2 = jnp.where(is1, NEG, logits)
        m2 = jnp.max(v2, axis=1, keepdims=True)
        i2 = jnp.min(jnp.where(v2 == m2, ii, E), axis=1, keepdims=True)
        is2 = ii == i2
        e = jnp.exp(m2 - m1)
        g1 = 1.0 / (1.0 + e)
        gates = jnp.where(is1, g1, jnp.where(is2, 1.0 - g1, 0.0))
        out_ref[...] = gates
        load_ref[...] = jnp.sum(gates, axis=0, keepdims=True)


def _bcast_argmax(v, iota):
    val, idx = v, iota
    for sh in (8, 4, 2, 1):
        perm = jnp.bitwise_xor(iota, sh)
        oval = jnp.take_along_axis(val, perm, axis=0)
        oidx = jnp.take_along_axis(idx, perm, axis=0)
        take = (oval > val) | ((oval == val) & (oidx < idx))
        val = jnp.where(take, oval, val)
        idx = jnp.where(take, oidx, idx)
    return val, idx


def _route_row(v, iota):
    m1, i1 = _bcast_argmax(v, iota)
    is1 = iota == i1
    v2 = jnp.where(is1, NEG, v)
    m2, i2 = _bcast_argmax(v2, iota)
    is2 = iota == i2
    e = jnp.exp(m2 - m1)
    g1 = 1.0 / (1.0 + e)
    return jnp.where(is1, g1, jnp.where(is2, 1.0 - g1, 0.0))


def _route_body(logits_hbm, gates_hbm, load_hbm, rows_v, out_v, all_v,
                loadout_v):
    s = lax.axis_index("s")
    base = s * RPS
    pltpu.sync_copy(logits_hbm.at[pl.ds(base, RPS)], rows_v)
    iota = lax.iota(jnp.int32, 16)
    for r in range(RPS):
        out_v[r] = _route_row(rows_v[r], iota)
    pltpu.sync_copy(out_v, gates_hbm.at[pl.ds(base, RPS)])

    @pl.when(s == 0)
    def _():
        pltpu.sync_copy(logits_hbm, all_v)
        acc = jnp.zeros((16,), jnp.float32)
        for b in range(B):
            acc = acc + _route_row(all_v[b], iota)
        loadout_v[...] = acc
        pltpu.sync_copy(loadout_v, load_hbm)


@functools.cache
def _route():
    return pl.kernel(
        _route_body,
        out_type=(jax.ShapeDtypeStruct((B, E), jnp.float32),
                  jax.ShapeDtypeStruct((E,), jnp.float32)),
        mesh=plsc.VectorSubcoreMesh(core_axis_name="c", subcore_axis_name="s",
                                    num_cores=1, num_subcores=NSUB),
        scratch_types=[
            pltpu.VMEM((RPS, 16), jnp.float32),
            pltpu.VMEM((RPS, 16), jnp.float32),
            pltpu.VMEM((B, 16), jnp.float32),
            pltpu.VMEM((16,), jnp.float32),
        ],
    )


def kernel(x, m, conv_w, conv_b, gate_w, gate_b, noise_w, noise_b):
    xt = jnp.transpose(x, (2, 3, 0, 1)).reshape(HW, B, C)
    mt = jnp.transpose(m, (2, 3, 0, 1)).reshape(HW, 1, B)
    cw = conv_w.reshape(1, 1)
    cb = conv_b.reshape(1, 1)
    gb = gate_b.reshape(1, E)
    nb = noise_b.reshape(1, E)
    eps = jax.random.normal(jax.random.key(1234), (B, E), jnp.float32)
    logits = pl.pallas_call(
        _logits_body,
        grid=(NSTEPS,),
        in_specs=[
            pl.BlockSpec(memory_space=pltpu.SMEM),
            pl.BlockSpec(memory_space=pltpu.SMEM),
            pl.BlockSpec((CHUNK, B, C), lambda i: (i, 0, 0)),
            pl.BlockSpec((CHUNK, 1, B), lambda i: (i, 0, 0)),
            pl.BlockSpec((E, C), lambda i: (0, 0)),
            pl.BlockSpec((1, E), lambda i: (0, 0)),
            pl.BlockSpec((E, C), lambda i: (0, 0)),
            pl.BlockSpec((1, E), lambda i: (0, 0)),
            pl.BlockSpec((B, E), lambda i: (0, 0)),
        ],
        out_specs=[pl.BlockSpec((B, E), lambda i: (0, 0)),
                   pl.BlockSpec((1, E), lambda i: (0, 0))],
        out_shape=[jax.ShapeDtypeStruct((B, E), jnp.float32),
                   jax.ShapeDtypeStruct((1, E), jnp.float32)],
        scratch_shapes=[pltpu.VMEM((B, C), jnp.float32)],
    )(cw, cb, xt, mt, gate_w, gb, noise_w, nb, eps)
    gates, load2 = logits
    return gates, load2.reshape(E)


# speedup vs baseline: 5.9815x; 1.0071x over previous
import functools

import jax
import jax.numpy as jnp
from jax import lax
from jax.experimental import pallas as pl
from jax.experimental.pallas import tpu as pltpu
from jax.experimental.pallas import tpu_sc as plsc

E = 16
B = 64
C = 384
HW = 196
BB = 8
NSUB = 16
RPS = B // NSUB
NEG = -3.0e38


CHUNK = 98
NSTEPS = HW // CHUNK


def _logits_body(cw_ref, cb_ref, xt_ref, mt_ref, gw_ref, gb_ref, nw_ref,
                 nb_ref, eps_ref, out_ref, load_ref, acc_ref):
    step = pl.program_id(0)
    scale = cw_ref[0, 0]
    bias = cb_ref[0, 0]
    wt = jnp.transpose(mt_ref[...], (0, 2, 1)) * scale + (1.0 + bias)
    part = jnp.sum(xt_ref[...] * wt, axis=0)

    @pl.when(step == 0)
    def _():
        acc_ref[...] = part

    @pl.when(step > 0)
    def _():
        acc_ref[...] += part

    @pl.when(step == NSTEPS - 1)
    def _():
        pooled = acc_ref[...] * (1.0 / HW)
        nt = (((1,), (1,)), ((), ()))
        clean = lax.dot_general(pooled, gw_ref[...], nt,
                                preferred_element_type=jnp.float32,
                                precision=lax.Precision.HIGHEST) + gb_ref[...]
        raw = lax.dot_general(pooled, nw_ref[...], nt,
                              preferred_element_type=jnp.float32,
                              precision=lax.Precision.HIGHEST) + nb_ref[...]
        std = (jnp.log(1.0 + jnp.exp(-jnp.abs(raw)))
               + jnp.maximum(raw, 0.0) + 0.01)
        logits = clean + eps_ref[...] * std
        ii = lax.broadcasted_iota(jnp.int32, (B, E), 1)
        m1 = jnp.max(logits, axis=1, keepdims=True)
        i1 = jnp.min(jnp.where(logits == m1, ii, E), axis=1, keepdims=True)
        is1 = ii == i1
        v2 = jnp.where(is1, NEG, logits)
        m2 = jnp.max(v2, axis=1, keepdims=True)
        i2 = jnp.min(jnp.where(v2 == m2, ii, E), axis=1, keepdims=True)
        is2 = ii == i2
        e = jnp.exp(m2 - m1)
        g1 = 1.0 / (1.0 + e)
        gates = jnp.where(is1, g1, jnp.where(is2, 1.0 - g1, 0.0))
        out_ref[...] = gates
        load_ref[...] = jnp.sum(gates, axis=0, keepdims=True)


def _bcast_argmax(v, iota):
    val, idx = v, iota
    for sh in (8, 4, 2, 1):
        perm = jnp.bitwise_xor(iota, sh)
        oval = jnp.take_along_axis(val, perm, axis=0)
        oidx = jnp.take_along_axis(idx, perm, axis=0)
        take = (oval > val) | ((oval == val) & (oidx < idx))
        val = jnp.where(take, oval, val)
        idx = jnp.where(take, oidx, idx)
    return val, idx


def _route_row(v, iota):
    m1, i1 = _bcast_argmax(v, iota)
    is1 = iota == i1
    v2 = jnp.where(is1, NEG, v)
    m2, i2 = _bcast_argmax(v2, iota)
    is2 = iota == i2
    e = jnp.exp(m2 - m1)
    g1 = 1.0 / (1.0 + e)
    return jnp.where(is1, g1, jnp.where(is2, 1.0 - g1, 0.0))


def _route_body(logits_hbm, gates_hbm, load_hbm, rows_v, out_v, all_v,
                loadout_v):
    s = lax.axis_index("s")
    base = s * RPS
    pltpu.sync_copy(logits_hbm.at[pl.ds(base, RPS)], rows_v)
    iota = lax.iota(jnp.int32, 16)
    for r in range(RPS):
        out_v[r] = _route_row(rows_v[r], iota)
    pltpu.sync_copy(out_v, gates_hbm.at[pl.ds(base, RPS)])

    @pl.when(s == 0)
    def _():
        pltpu.sync_copy(logits_hbm, all_v)
        acc = jnp.zeros((16,), jnp.float32)
        for b in range(B):
            acc = acc + _route_row(all_v[b], iota)
        loadout_v[...] = acc
        pltpu.sync_copy(loadout_v, load_hbm)


@functools.cache
def _route():
    return pl.kernel(
        _route_body,
        out_type=(jax.ShapeDtypeStruct((B, E), jnp.float32),
                  jax.ShapeDtypeStruct((E,), jnp.float32)),
        mesh=plsc.VectorSubcoreMesh(core_axis_name="c", subcore_axis_name="s",
                                    num_cores=1, num_subcores=NSUB),
        scratch_types=[
            pltpu.VMEM((RPS, 16), jnp.float32),
            pltpu.VMEM((RPS, 16), jnp.float32),
            pltpu.VMEM((B, 16), jnp.float32),
            pltpu.VMEM((16,), jnp.float32),
        ],
    )


def kernel(x, m, conv_w, conv_b, gate_w, gate_b, noise_w, noise_b):
    xt = jnp.transpose(x, (2, 3, 0, 1)).reshape(HW, B, C)
    mt = jnp.transpose(m, (2, 3, 0, 1)).reshape(HW, 1, B)
    cw = conv_w.reshape(1, 1)
    cb = conv_b.reshape(1, 1)
    gb = gate_b.reshape(1, E)
    nb = noise_b.reshape(1, E)
    eps = jax.random.normal(jax.random.key(1234), (B, E), jnp.float32)
    logits = pl.pallas_call(
        _logits_body,
        grid=(NSTEPS,),
        in_specs=[
            pl.BlockSpec(memory_space=pltpu.SMEM),
            pl.BlockSpec(memory_space=pltpu.SMEM),
            pl.BlockSpec((CHUNK, B, C), lambda i: (i, 0, 0)),
            pl.BlockSpec((CHUNK, 1, B), lambda i: (i, 0, 0)),
            pl.BlockSpec((E, C), lambda i: (0, 0)),
            pl.BlockSpec((1, E), lambda i: (0, 0)),
            pl.BlockSpec((E, C), lambda i: (0, 0)),
            pl.BlockSpec((1, E), lambda i: (0, 0)),
            pl.BlockSpec((B, E), lambda i: (0, 0)),
        ],
        out_specs=[pl.BlockSpec((B, E), lambda i: (0, 0)),
                   pl.BlockSpec((1, E), lambda i: (0, 0))],
        out_shape=[jax.ShapeDtypeStruct((B, E), jnp.float32),
                   jax.ShapeDtypeStruct((1, E), jnp.float32)],
        scratch_shapes=[pltpu.VMEM((B, C), jnp.float32)],
    )(cw, cb, xt, mt, gate_w, gb, noise_w, nb, eps)
    gates, load2 = logits
    return gates, load2.reshape(E)
